# baseline (speedup 1.0000x reference)
import jax
import jax.numpy as jnp
from jax import lax
from jax.experimental import pallas as pl
from jax.experimental.pallas import tpu as pltpu
from jax.experimental.pallas import tpu_sc as plsc

_N = 10000
_NP = 10240
_E = 320000
_D = 128
_NS = 16
_CH = 50
_ER = _E // _CH
_RPT = _ER // _NS
_KC = 40
_NBUF = 5
_NPT = _NP // _NS
_ZW = 64

_MESH = plsc.VectorSubcoreMesh(core_axis_name="c", subcore_axis_name="s",
                               num_cores=1)


def _make_scatter():
  outs = jax.ShapeDtypeStruct((_NP, _D), jnp.float32)
  scratch = [
      pltpu.VMEM((_KC, _CH), jnp.int32),
      pltpu.VMEM((_KC, _CH), jnp.int32),
  ]
  scratch += [pltpu.VMEM((_CH, _D), jnp.float32) for _ in range(_NBUF)]
  scratch += [pltpu.VMEM_SHARED((_NP, _D), jnp.float32)]
  scratch += [pltpu.SemaphoreType.DMA for _ in range(_NBUF)]

  def body(z, srch, dsth, aggo, srcv, dstv, b0, b1, b2, b3, b4, aggs,
           s0, s1, s2, s3, s4):
    bufs = (b0, b1, b2, b3, b4)
    sems = (s0, s1, s2, s3, s4)
    sid = lax.axis_index("s")

    z16 = jnp.zeros((16,), jnp.float32)

    @pl.loop(0, _ZW)
    def _zero(i):
      for j in range(_D // 16):
        b0[i, pl.ds(j * 16, 16)] = z16

    zb = b0.at[pl.ds(0, _ZW)]
    row0 = sid * _NPT
    for t in range(_NPT // _ZW):
      pltpu.sync_copy(zb, aggs.at[pl.ds(row0 + t * _ZW, _ZW)])
    plsc.subcore_barrier()

    @pl.loop(0, _RPT, step=_KC)
    def _outer(kk):
      pltpu.sync_copy(srch.at[pl.ds(sid * _RPT + kk, _KC)], srcv)
      pltpu.sync_copy(dsth.at[pl.ds(sid * _RPT + kk, _KC)], dstv)

      @pl.loop(0, _KC, step=_NBUF)
      def _main(jj):
        descs = []
        for b in range(_NBUF):
          descs.append(
              pltpu.async_copy(z.at[srcv.at[jj + b]], bufs[b], sems[b]))
        for b in range(_NBUF):
          descs[b].wait()
          pltpu.sync_copy(bufs[b], aggs.at[dstv.at[jj + b]], add=True)

    plsc.subcore_barrier()
    for t in range(_NPT // _ZW):
      pltpu.sync_copy(aggs.at[pl.ds(row0 + t * _ZW, _ZW)], zb)
      pltpu.sync_copy(zb, aggo.at[pl.ds(row0 + t * _ZW, _ZW)])

  return pl.kernel(body, out_type=outs, mesh=_MESH,
                   scratch_types=tuple(scratch))


_BR = 1000
_G = _N // _BR
_F32 = dict(preferred_element_type=jnp.float32,
            precision=lax.Precision.HIGHEST)


def _combine(h, Ws, Wn, b, q, degq, act):
  Din, Dout = Ws.shape

  def body(h_ref, ws_ref, wn_ref, b_ref, q_ref, d_ref, o_ref):
    invd = 1.0 / jnp.maximum(d_ref[:, 0:1], 1.0)
    t = jnp.dot(h_ref[...], ws_ref[...], **_F32)
    t = t + jnp.dot(q_ref[...] * invd, wn_ref[...], **_F32) + b_ref[...]
    if act:
      t = jnp.maximum(t, 0.0)
    o_ref[...] = t

  return pl.pallas_call(
      body,
      grid=(_G,),
      in_specs=[
          pl.BlockSpec((_BR, Din), lambda i: (i, 0)),
          pl.BlockSpec((Din, Dout), lambda i: (0, 0)),
          pl.BlockSpec((Din, Dout), lambda i: (0, 0)),
          pl.BlockSpec((1, Dout), lambda i: (0, 0)),
          pl.BlockSpec((_BR, _D), lambda i: (i, 0)),
          pl.BlockSpec((_BR, _D), lambda i: (i, 0)),
      ],
      out_specs=pl.BlockSpec((_BR, Dout), lambda i: (i, 0)),
      out_shape=jax.ShapeDtypeStruct((_N, Dout), jnp.float32),
  )(h, Ws, Wn, b, q, degq)


def kernel(edge_index, inputs, W0s, W0n, b0, W1s, W1n, b1, W2s, W2n, b2):
  ei = edge_index.astype(jnp.int32)
  src = ei[0].reshape(_ER, _CH)
  dst = ei[1].reshape(_ER, _CH)

  b0r = b0.reshape(1, -1)
  b1r = b1.reshape(1, -1)
  b2r = b2.reshape(1, -1)
  ones = jnp.ones((_N, _D), jnp.float32)
  src0 = jnp.zeros_like(src)

  scat = _make_scatter()

  degq = scat(ones, src0, dst)
  q = scat(inputs, src, dst)
  h1 = _combine(inputs, W0s, W0n, b0r, q, degq, False)
  q = scat(h1, src, dst)
  h2 = _combine(h1, W1s, W1n, b1r, q, degq, True)
  q = scat(h2, src, dst)
  h3 = _combine(h2, W1s, W1n, b1r, q, degq, True)
  q = scat(h3, src, dst)
  out = _combine(h3, W2s, W2n, b2r, q, degq, False)
  return out, h3


# speedup vs baseline: 8.3305x; 8.3305x over previous
import jax
import jax.numpy as jnp
from jax import lax
from jax.experimental import pallas as pl
from jax.experimental.pallas import tpu as pltpu
from jax.experimental.pallas import tpu_sc as plsc

_N = 10000
_NP = 10240
_E = 320000
_D = 128
_NS = 16
_CH = 50
_ER = _E // _CH
_RPT = _ER // _NS
_KC = 40
_NBUF = 5
_NPT = _NP // _NS
_ZW = 64

_MESH = plsc.VectorSubcoreMesh(core_axis_name="c", subcore_axis_name="s",
                               num_cores=1)


def _make_scatter():
  outs = jax.ShapeDtypeStruct((_NP, _D), jnp.float32)
  scratch = [
      pltpu.VMEM((_KC, _CH), jnp.int32),
      pltpu.VMEM((_KC, _CH), jnp.int32),
  ]
  scratch += [pltpu.VMEM((_CH, _D), jnp.float32) for _ in range(_NBUF)]
  scratch += [pltpu.VMEM_SHARED((_NP, _D), jnp.float32)]
  scratch += [pltpu.SemaphoreType.DMA for _ in range(_NBUF)]

  def body(z, srch, dsth, aggo, srcv, dstv, b0, b1, b2, b3, b4, aggs,
           s0, s1, s2, s3, s4):
    bufs = (b0, b1, b2, b3, b4)
    sems = (s0, s1, s2, s3, s4)
    sid = lax.axis_index("s")

    z16 = jnp.zeros((16,), jnp.float32)

    @pl.loop(0, _ZW)
    def _zero(i):
      for j in range(_D // 16):
        b0[i, pl.ds(j * 16, 16)] = z16

    zb = b0.at[pl.ds(0, _ZW)]
    row0 = sid * _NPT
    for t in range(_NPT // _ZW):
      pltpu.sync_copy(zb, aggs.at[pl.ds(row0 + t * _ZW, _ZW)])
    plsc.subcore_barrier()

    @pl.loop(0, _RPT, step=_KC)
    def _outer(kk):
      pltpu.sync_copy(srch.at[pl.ds(sid * _RPT + kk, _KC)], srcv)
      pltpu.sync_copy(dsth.at[pl.ds(sid * _RPT + kk, _KC)], dstv)

      @pl.loop(0, _KC, step=_NBUF)
      def _main(jj):
        descs = []
        for b in range(_NBUF):
          descs.append(
              pltpu.async_copy(z.at[srcv.at[jj + b]], bufs[b], sems[b]))
        for b in range(_NBUF):
          descs[b].wait()
          pltpu.sync_copy(bufs[b], aggs.at[dstv.at[jj + b]], add=True)

    plsc.subcore_barrier()
    for t in range(_NPT // _ZW):
      pltpu.sync_copy(aggs.at[pl.ds(row0 + t * _ZW, _ZW)], zb)
      pltpu.sync_copy(zb, aggo.at[pl.ds(row0 + t * _ZW, _ZW)])

  return pl.kernel(body, out_type=outs, mesh=_MESH,
                   scratch_types=tuple(scratch))


_BR = 1000
_G = _N // _BR
_F32 = dict(preferred_element_type=jnp.float32,
            precision=lax.Precision.HIGHEST)


def _combine(h, Ws, Wn, b, q, degq, act):
  Din, Dout = Ws.shape

  def body(h_ref, ws_ref, wn_ref, b_ref, q_ref, d_ref, o_ref):
    invd = 1.0 / jnp.maximum(d_ref[:, 0:1], 1.0)
    t = jnp.dot(h_ref[...], ws_ref[...], **_F32)
    t = t + jnp.dot(q_ref[...] * invd, wn_ref[...], **_F32) + b_ref[...]
    if act:
      t = jnp.maximum(t, 0.0)
    o_ref[...] = t

  return pl.pallas_call(
      body,
      grid=(_G,),
      in_specs=[
          pl.BlockSpec((_BR, Din), lambda i: (i, 0)),
          pl.BlockSpec((Din, Dout), lambda i: (0, 0)),
          pl.BlockSpec((Din, Dout), lambda i: (0, 0)),
          pl.BlockSpec((1, Dout), lambda i: (0, 0)),
          pl.BlockSpec((_BR, _D), lambda i: (i, 0)),
          pl.BlockSpec((_BR, _D), lambda i: (i, 0)),
      ],
      out_specs=pl.BlockSpec((_BR, Dout), lambda i: (i, 0)),
      out_shape=jax.ShapeDtypeStruct((_N, Dout), jnp.float32),
  )(h, Ws, Wn, b, q, degq)


def kernel(edge_index, inputs, W0s, W0n, b0, W1s, W1n, b1, W2s, W2n, b2):
  ei = edge_index.astype(jnp.int32)
  src = ei[0].reshape(_ER, _CH)
  dst = ei[1].reshape(_ER, _CH)

  b0r = b0.reshape(1, -1)
  b1r = b1.reshape(1, -1)
  b2r = b2.reshape(1, -1)
  ones = jnp.ones((_N, _D), jnp.float32)

  scat = _make_scatter()

  degq = scat(ones, src, dst)
  q = scat(inputs, src, dst)
  h1 = _combine(inputs, W0s, W0n, b0r, q, degq, False)
  q = scat(h1, src, dst)
  h2 = _combine(h1, W1s, W1n, b1r, q, degq, True)
  q = scat(h2, src, dst)
  h3 = _combine(h2, W1s, W1n, b1r, q, degq, True)
  q = scat(h3, src, dst)
  out = _combine(h3, W2s, W2n, b2r, q, degq, False)
  return out, h3


# speedup vs baseline: 11.9405x; 1.4333x over previous
import jax
import jax.numpy as jnp
from jax import lax
from jax.experimental import pallas as pl
from jax.experimental.pallas import tpu as pltpu
from jax.experimental.pallas import tpu_sc as plsc

_N = 10000
_NP = 10240
_E = 320000
_D = 128
_NS = 16
_CH = 125
_ER = _E // _CH
_RPT = _ER // _NS
_KC = 32
_NBUF = 2
_NPT = _NP // _NS
_ZW = 64

_MESH = plsc.VectorSubcoreMesh(core_axis_name="c", subcore_axis_name="s",
                               num_cores=1)


def _make_scatter():
  outs = jax.ShapeDtypeStruct((_NP, _D), jnp.float32)
  scratch = [
      pltpu.VMEM((_KC, _CH), jnp.int32),
      pltpu.VMEM((_KC, _CH), jnp.int32),
  ]
  scratch += [pltpu.VMEM((_CH, _D), jnp.float32) for _ in range(_NBUF)]
  scratch += [pltpu.VMEM_SHARED((_NP, _D), jnp.float32)]
  scratch += [pltpu.SemaphoreType.DMA for _ in range(_NBUF)]

  def body(z, srch, dsth, aggo, srcv, dstv, b0, b1, aggs, s0, s1):
    bufs = (b0, b1)
    sems = (s0, s1)
    sid = lax.axis_index("s")

    z16 = jnp.zeros((16,), jnp.float32)

    @pl.loop(0, _ZW)
    def _zero(i):
      for j in range(_D // 16):
        b0[i, pl.ds(j * 16, 16)] = z16

    zb = b0.at[pl.ds(0, _ZW)]
    row0 = sid * _NPT
    for t in range(_NPT // _ZW):
      pltpu.sync_copy(zb, aggs.at[pl.ds(row0 + t * _ZW, _ZW)])
    plsc.subcore_barrier()

    @pl.loop(0, _RPT, step=_KC)
    def _outer(kk):
      pltpu.sync_copy(srch.at[pl.ds(sid * _RPT + kk, _KC)], srcv)
      pltpu.sync_copy(dsth.at[pl.ds(sid * _RPT + kk, _KC)], dstv)

      for b in range(_NBUF):
        pltpu.async_copy(z.at[srcv.at[b]], bufs[b], sems[b])

      @pl.loop(0, _KC, step=_NBUF)
      def _main(jj):
        for b in range(_NBUF):
          j = jj + b
          pltpu.make_async_copy(z.at[srcv.at[j]], bufs[b], sems[b]).wait()
          pltpu.sync_copy(bufs[b], aggs.at[dstv.at[j]], add=True)

          @pl.when(j + _NBUF < _KC)
          def _prefetch():
            pltpu.async_copy(z.at[srcv.at[j + _NBUF]], bufs[b], sems[b])

    plsc.subcore_barrier()
    for t in range(_NPT // _ZW):
      pltpu.sync_copy(aggs.at[pl.ds(row0 + t * _ZW, _ZW)], zb)
      pltpu.sync_copy(zb, aggo.at[pl.ds(row0 + t * _ZW, _ZW)])

  return pl.kernel(body, out_type=outs, mesh=_MESH,
                   scratch_types=tuple(scratch))


_BR = 1000
_G = _N // _BR
_F32 = dict(preferred_element_type=jnp.float32,
            precision=lax.Precision.HIGHEST)


def _combine(h, Ws, Wn, b, q, degq, act):
  Din, Dout = Ws.shape

  def body(h_ref, ws_ref, wn_ref, b_ref, q_ref, d_ref, o_ref):
    invd = 1.0 / jnp.maximum(d_ref[:, 0:1], 1.0)
    t = jnp.dot(h_ref[...], ws_ref[...], **_F32)
    t = t + jnp.dot(q_ref[...] * invd, wn_ref[...], **_F32) + b_ref[...]
    if act:
      t = jnp.maximum(t, 0.0)
    o_ref[...] = t

  return pl.pallas_call(
      body,
      grid=(_G,),
      in_specs=[
          pl.BlockSpec((_BR, Din), lambda i: (i, 0)),
          pl.BlockSpec((Din, Dout), lambda i: (0, 0)),
          pl.BlockSpec((Din, Dout), lambda i: (0, 0)),
          pl.BlockSpec((1, Dout), lambda i: (0, 0)),
          pl.BlockSpec((_BR, _D), lambda i: (i, 0)),
          pl.BlockSpec((_BR, _D), lambda i: (i, 0)),
      ],
      out_specs=pl.BlockSpec((_BR, Dout), lambda i: (i, 0)),
      out_shape=jax.ShapeDtypeStruct((_N, Dout), jnp.float32),
  )(h, Ws, Wn, b, q, degq)


def kernel(edge_index, inputs, W0s, W0n, b0, W1s, W1n, b1, W2s, W2n, b2):
  ei = edge_index.astype(jnp.int32)
  src = ei[0].reshape(_ER, _CH)
  dst = ei[1].reshape(_ER, _CH)

  b0r = b0.reshape(1, -1)
  b1r = b1.reshape(1, -1)
  b2r = b2.reshape(1, -1)
  ones = jnp.ones((_N, _D), jnp.float32)

  scat = _make_scatter()

  degq = scat(ones, src, dst)
  q = scat(inputs, src, dst)
  h1 = _combine(inputs, W0s, W0n, b0r, q, degq, False)
  q = scat(h1, src, dst)
  h2 = _combine(h1, W1s, W1n, b1r, q, degq, True)
  q = scat(h2, src, dst)
  h3 = _combine(h2, W1s, W1n, b1r, q, degq, True)
  q = scat(h3, src, dst)
  out = _combine(h3, W2s, W2n, b2r, q, degq, False)
  return out, h3
